# Initial kernel scaffold; baseline (speedup 1.0000x reference)
import jax
import jax.numpy as jnp
from jax.experimental import pallas as pl


def kernel(text_embeddings, label_embeddings, target_labels, W1, b1, W2, b2):
    raise NotImplementedError("write your pallas kernel here")


# speedup vs baseline: 6.0901x; 6.0901x over previous
import numpy as np
import jax
import jax.numpy as jnp
from jax.experimental import pallas as pl
from jax.experimental.pallas import tpu as pltpu

NEG_FILL = -100.0
INV_TEMP = 1.0 / 0.07


def _f32_key(x):
    i = np.array(x, np.float32).view(np.int32)
    return int(np.bitwise_xor(i, np.bitwise_and(np.right_shift(i, 31),
                                                np.int32(0x7FFFFFFF))))


_KEY_NINF = _f32_key(NEG_FILL)
_KEY_LO = _f32_key(-101.0)
_KEY_HI = _f32_key(2.0)


def _l2n(x):
    nrm = jnp.sqrt(jnp.sum(x * x, axis=-1, keepdims=True))
    return x / jnp.maximum(nrm, 1e-12)


def _proj_normed(x_f32, w1_ref, b1_ref, w2_ref, b2_ref):
    xn = _l2n(x_f32).astype(jnp.bfloat16)
    h = jnp.dot(xn, w1_ref[...], preferred_element_type=jnp.float32) + b1_ref[...]
    h = jnp.maximum(h, 0.0).astype(jnp.bfloat16)
    p = jnp.dot(h, w2_ref[...], preferred_element_type=jnp.float32) + b2_ref[...]
    return _l2n(p)


def _label_kernel(lab_ref, w1_ref, b1_ref, w2_ref, b2_ref, out_ref):
    out_ref[...] = _proj_normed(lab_ref[...], w1_ref, b1_ref, w2_ref,
                                b2_ref).astype(jnp.bfloat16)


def _loss_kernel(txt_ref, tgt_ref, w1_ref, b1_ref, w2_ref, b2_ref, ln_ref,
                 out_ref):
    pn = _proj_normed(txt_ref[...], w1_ref, b1_ref, w2_ref,
                      b2_ref).astype(jnp.bfloat16)
    sim = jax.lax.dot_general(pn, ln_ref[...], (((1,), (1,)), ((), ())),
                              preferred_element_type=jnp.float32)

    pos = tgt_ref[...] > 0
    n_pos = jnp.sum(jnp.where(pos, jnp.int32(1), 0), axis=-1, keepdims=True)

    ib = jax.lax.bitcast_convert_type(sim, jnp.int32)
    key = ib ^ ((ib >> 31) & jnp.int32(0x7FFFFFFF))
    mkey = jnp.where(pos, jnp.int32(_KEY_NINF), key)

    def _count_ge(t):
        return jnp.sum(jnp.where(mkey >= t, jnp.int32(1), 0), axis=-1,
                       keepdims=True)

    zero = jnp.zeros_like(n_pos)
    ge0 = _count_ge(zero) >= n_pos
    lo = jnp.where(ge0, zero, jnp.int32(_KEY_LO))
    hi = jnp.where(ge0, jnp.int32(_KEY_HI), zero)

    def body(_, carry):
        lo, hi = carry
        mid = lo + ((hi - lo) >> 1)
        ge = _count_ge(mid) >= n_pos
        return jnp.where(ge, mid, lo), jnp.where(ge, hi, mid)

    lo, _ = jax.lax.fori_loop(0, 31, body, (lo, hi))

    s = sim * INV_TEMP
    es = jnp.exp(s)
    sel = pos | (mkey >= lo)
    denom = jnp.sum(jnp.where(sel, es, 0.0), axis=-1, keepdims=True)
    sum_pos_s = jnp.sum(jnp.where(pos, s, 0.0), axis=-1, keepdims=True)
    loss_rows = jnp.log(denom) - sum_pos_s / n_pos.astype(jnp.float32)
    out_ref[...] = jnp.zeros((1, 1, 128), jnp.float32) + jnp.sum(loss_rows)


def _full(shape):
    return pl.BlockSpec(shape, lambda *_: tuple(0 for _ in shape))


def kernel(text_embeddings, label_embeddings, target_labels, W1, b1, W2, b2):
    B, D = text_embeddings.shape
    L = label_embeddings.shape[0]
    H = W1.shape[1]
    blk = 256 if B % 256 == 0 else B
    lblk = 512 if L % 512 == 0 else L
    nblk = B // blk

    w1b = W1.astype(jnp.bfloat16)
    w2b = W2.astype(jnp.bfloat16)
    b1r = b1.reshape(1, H)
    b2r = b2.reshape(1, D)

    ln = pl.pallas_call(
        _label_kernel,
        grid=(L // lblk,),
        in_specs=[
            pl.BlockSpec((lblk, D), lambda i: (i, 0)),
            _full((D, H)), _full((1, H)), _full((H, D)), _full((1, D)),
        ],
        out_specs=pl.BlockSpec((lblk, D), lambda i: (i, 0)),
        out_shape=jax.ShapeDtypeStruct((L, D), jnp.bfloat16),
        compiler_params=pltpu.CompilerParams(
            dimension_semantics=("parallel",),
            vmem_limit_bytes=52 * 1024 * 1024,
        ),
        name="tla_label_proj",
    )(label_embeddings, w1b, b1r, w2b, b2r)

    partials = pl.pallas_call(
        _loss_kernel,
        grid=(nblk,),
        in_specs=[
            pl.BlockSpec((blk, D), lambda i: (i, 0)),
            pl.BlockSpec((blk, L), lambda i: (i, 0)),
            _full((D, H)), _full((1, H)), _full((H, D)), _full((1, D)),
            _full((L, D)),
        ],
        out_specs=pl.BlockSpec((1, 1, 128), lambda i: (i, 0, 0)),
        out_shape=jax.ShapeDtypeStruct((nblk, 1, 128), jnp.float32),
        compiler_params=pltpu.CompilerParams(
            dimension_semantics=("parallel",),
            vmem_limit_bytes=52 * 1024 * 1024,
        ),
        name="tla_loss",
    )(text_embeddings, target_labels, w1b, b1r, w2b, b2r, ln)

    return jnp.sum(partials[:, 0, 0]) / B


# speedup vs baseline: 6.6471x; 1.0915x over previous
import numpy as np
import jax
import jax.numpy as jnp
from jax.experimental import pallas as pl
from jax.experimental.pallas import tpu as pltpu

NEG_FILL = -100.0
INV_TEMP = 1.0 / 0.07


def _f32_key(x):
    i = np.array(x, np.float32).view(np.int32)
    return int(np.bitwise_xor(i, np.bitwise_and(np.right_shift(i, 31),
                                                np.int32(0x7FFFFFFF))))


_KEY_NINF = _f32_key(NEG_FILL)
_KEY_LO = _f32_key(-101.0)
_KEY_HI = _f32_key(2.0)


def _l2n(x):
    nrm = jnp.sqrt(jnp.sum(x * x, axis=-1, keepdims=True))
    return x / jnp.maximum(nrm, 1e-12)


def _proj_normed(x_f32, w1_ref, b1_ref, w2_ref, b2_ref):
    xn = _l2n(x_f32).astype(jnp.bfloat16)
    h = jnp.dot(xn, w1_ref[...], preferred_element_type=jnp.float32) + b1_ref[...]
    h = jnp.maximum(h, 0.0).astype(jnp.bfloat16)
    p = jnp.dot(h, w2_ref[...], preferred_element_type=jnp.float32) + b2_ref[...]
    return _l2n(p)


def _label_kernel(lab_ref, w1_ref, b1_ref, w2_ref, b2_ref, out_ref):
    out_ref[...] = _proj_normed(lab_ref[...], w1_ref, b1_ref, w2_ref,
                                b2_ref).astype(jnp.bfloat16)


def _loss_kernel(txt_ref, tgt_ref, w1_ref, b1_ref, w2_ref, b2_ref, ln_ref,
                 out_ref):
    pn = _proj_normed(txt_ref[...], w1_ref, b1_ref, w2_ref,
                      b2_ref).astype(jnp.bfloat16)
    sim = jax.lax.dot_general(pn, ln_ref[...], (((1,), (1,)), ((), ())),
                              preferred_element_type=jnp.float32)

    pos = tgt_ref[...] > 0
    n_pos = jnp.sum(jnp.where(pos, jnp.int32(1), 0), axis=-1, keepdims=True)

    ib = jax.lax.bitcast_convert_type(sim, jnp.int32)
    key = ib ^ ((ib >> 31) & jnp.int32(0x7FFFFFFF))
    mkey = jnp.where(pos, jnp.int32(_KEY_NINF), key)

    def _count_ge(t):
        return jnp.sum(jnp.where(mkey >= t, jnp.int32(1), 0), axis=-1,
                       keepdims=True)

    zero = jnp.zeros_like(n_pos)
    ge0 = _count_ge(zero) >= n_pos
    lo = jnp.where(ge0, zero, jnp.int32(_KEY_LO))
    hi = jnp.where(ge0, jnp.int32(_KEY_HI), zero)

    def body(_, carry):
        lo, hi = carry
        mid = lo + ((hi - lo) >> 1)
        ge = _count_ge(mid) >= n_pos
        return jnp.where(ge, mid, lo), jnp.where(ge, hi, mid)

    lo, _ = jax.lax.fori_loop(0, 31, body, (lo, hi))

    s = sim * INV_TEMP
    es = jnp.exp(s)
    sel = pos | (mkey >= lo)
    denom = jnp.sum(jnp.where(sel, es, 0.0), axis=-1, keepdims=True)
    sum_pos_s = jnp.sum(jnp.where(pos, s, 0.0), axis=-1, keepdims=True)
    loss_rows = jnp.log(denom) - sum_pos_s / n_pos.astype(jnp.float32)
    out_ref[...] = jnp.zeros((1, 1, 128), jnp.float32) + jnp.sum(loss_rows)


def _full(shape):
    return pl.BlockSpec(shape, lambda *_: tuple(0 for _ in shape))


def kernel(text_embeddings, label_embeddings, target_labels, W1, b1, W2, b2):
    B, D = text_embeddings.shape
    L = label_embeddings.shape[0]
    H = W1.shape[1]
    blk = 512 if B % 512 == 0 else B
    lblk = 512 if L % 512 == 0 else L
    nblk = B // blk

    w1b = W1.astype(jnp.bfloat16)
    w2b = W2.astype(jnp.bfloat16)
    b1r = b1.reshape(1, H)
    b2r = b2.reshape(1, D)

    ln = pl.pallas_call(
        _label_kernel,
        grid=(L // lblk,),
        in_specs=[
            pl.BlockSpec((lblk, D), lambda i: (i, 0)),
            _full((D, H)), _full((1, H)), _full((H, D)), _full((1, D)),
        ],
        out_specs=pl.BlockSpec((lblk, D), lambda i: (i, 0)),
        out_shape=jax.ShapeDtypeStruct((L, D), jnp.bfloat16),
        compiler_params=pltpu.CompilerParams(
            dimension_semantics=("parallel",),
            vmem_limit_bytes=52 * 1024 * 1024,
        ),
        name="tla_label_proj",
    )(label_embeddings, w1b, b1r, w2b, b2r)

    partials = pl.pallas_call(
        _loss_kernel,
        grid=(nblk,),
        in_specs=[
            pl.BlockSpec((blk, D), lambda i: (i, 0)),
            pl.BlockSpec((blk, L), lambda i: (i, 0)),
            _full((D, H)), _full((1, H)), _full((H, D)), _full((1, D)),
            _full((L, D)),
        ],
        out_specs=pl.BlockSpec((1, 1, 128), lambda i: (i, 0, 0)),
        out_shape=jax.ShapeDtypeStruct((nblk, 1, 128), jnp.float32),
        compiler_params=pltpu.CompilerParams(
            dimension_semantics=("parallel",),
            vmem_limit_bytes=52 * 1024 * 1024,
        ),
        name="tla_loss",
    )(text_embeddings, target_labels, w1b, b1r, w2b, b2r, ln)

    return jnp.sum(partials[:, 0, 0]) / B


# speedup vs baseline: 11.2234x; 1.6885x over previous
import numpy as np
import jax
import jax.numpy as jnp
from jax.experimental import pallas as pl
from jax.experimental.pallas import tpu as pltpu

NEG_FILL = -100.0
BELOW = -200.0
ABOVE = 2.0
INV_TEMP = 1.0 / 0.07


def _l2n(x):
    nrm = jnp.sqrt(jnp.sum(x * x, axis=-1, keepdims=True))
    return x / jnp.maximum(nrm, 1e-12)


def _proj_normed(x_f32, w1_ref, b1_ref, w2_ref, b2_ref):
    xn = _l2n(x_f32).astype(jnp.bfloat16)
    h = jnp.dot(xn, w1_ref[...], preferred_element_type=jnp.float32) + b1_ref[...]
    h = jnp.maximum(h, 0.0).astype(jnp.bfloat16)
    p = jnp.dot(h, w2_ref[...], preferred_element_type=jnp.float32) + b2_ref[...]
    return _l2n(p)


def _label_kernel(lab_ref, w1_ref, b1_ref, w2_ref, b2_ref, out_ref):
    out_ref[...] = _proj_normed(lab_ref[...], w1_ref, b1_ref, w2_ref,
                                b2_ref).astype(jnp.bfloat16)


def _loss_kernel(txt_ref, tgt_ref, w1_ref, b1_ref, w2_ref, b2_ref, ln_ref,
                 out_ref):
    pn = _proj_normed(txt_ref[...], w1_ref, b1_ref, w2_ref,
                      b2_ref).astype(jnp.bfloat16)
    sim = jax.lax.dot_general(pn, ln_ref[...], (((1,), (1,)), ((), ())),
                              preferred_element_type=jnp.float32)

    tgt = tgt_ref[...]
    pos = tgt > 0
    n_pos = jnp.sum(tgt, axis=-1, keepdims=True)
    masked = jnp.where(pos, NEG_FILL, sim)

    max_np = jnp.max(n_pos)
    t0 = jnp.full_like(n_pos, ABOVE, dtype=jnp.float32)
    thr0 = jnp.full_like(n_pos, BELOW, dtype=jnp.float32)

    def cond(carry):
        return carry[0] < max_np

    def body(carry):
        i, t, thr = carry
        m = jnp.max(jnp.where(masked < t, masked, BELOW), axis=-1,
                    keepdims=True)
        thr = jnp.where(i == n_pos - 1, m, thr)
        return i + 1, m, thr

    _, _, thr = jax.lax.while_loop(cond, body, (jnp.int32(0), t0, thr0))

    s = sim * INV_TEMP
    es = jnp.exp(s)
    sel = pos | (masked >= thr)
    denom = jnp.sum(jnp.where(sel, es, 0.0), axis=-1, keepdims=True)
    sum_pos_s = jnp.sum(jnp.where(pos, s, 0.0), axis=-1, keepdims=True)
    loss_rows = jnp.log(denom) - sum_pos_s / n_pos.astype(jnp.float32)
    out_ref[...] = jnp.zeros((1, 1, 128), jnp.float32) + jnp.sum(loss_rows)


def _full(shape):
    return pl.BlockSpec(shape, lambda *_: tuple(0 for _ in shape))


def kernel(text_embeddings, label_embeddings, target_labels, W1, b1, W2, b2):
    B, D = text_embeddings.shape
    L = label_embeddings.shape[0]
    H = W1.shape[1]
    blk = 512 if B % 512 == 0 else B
    lblk = 512 if L % 512 == 0 else L
    nblk = B // blk

    w1b = W1.astype(jnp.bfloat16)
    w2b = W2.astype(jnp.bfloat16)
    b1r = b1.reshape(1, H)
    b2r = b2.reshape(1, D)

    ln = pl.pallas_call(
        _label_kernel,
        grid=(L // lblk,),
        in_specs=[
            pl.BlockSpec((lblk, D), lambda i: (i, 0)),
            _full((D, H)), _full((1, H)), _full((H, D)), _full((1, D)),
        ],
        out_specs=pl.BlockSpec((lblk, D), lambda i: (i, 0)),
        out_shape=jax.ShapeDtypeStruct((L, D), jnp.bfloat16),
        compiler_params=pltpu.CompilerParams(
            dimension_semantics=("parallel",),
            vmem_limit_bytes=52 * 1024 * 1024,
        ),
        name="tla_label_proj",
    )(label_embeddings, w1b, b1r, w2b, b2r)

    partials = pl.pallas_call(
        _loss_kernel,
        grid=(nblk,),
        in_specs=[
            pl.BlockSpec((blk, D), lambda i: (i, 0)),
            pl.BlockSpec((blk, L), lambda i: (i, 0)),
            _full((D, H)), _full((1, H)), _full((H, D)), _full((1, D)),
            _full((L, D)),
        ],
        out_specs=pl.BlockSpec((1, 1, 128), lambda i: (i, 0, 0)),
        out_shape=jax.ShapeDtypeStruct((nblk, 1, 128), jnp.float32),
        compiler_params=pltpu.CompilerParams(
            dimension_semantics=("parallel",),
            vmem_limit_bytes=52 * 1024 * 1024,
        ),
        name="tla_loss",
    )(text_embeddings, target_labels, w1b, b1r, w2b, b2r, ln)

    return jnp.sum(partials[:, 0, 0]) / B


# speedup vs baseline: 11.3910x; 1.0149x over previous
import functools

import jax
import jax.numpy as jnp
from jax.experimental import pallas as pl
from jax.experimental.pallas import tpu as pltpu

NEG_FILL = -100.0
BELOW = -200.0
ABOVE = 2.0
INV_TEMP = 1.0 / 0.07


def _l2n(x):
    nrm = jnp.sqrt(jnp.sum(x * x, axis=-1, keepdims=True))
    return x / jnp.maximum(nrm, 1e-12)


def _proj_normed(x_f32, w1_ref, b1_ref, w2_ref, b2_ref):
    xn = _l2n(x_f32).astype(jnp.bfloat16)
    h = jnp.dot(xn, w1_ref[...], preferred_element_type=jnp.float32) + b1_ref[...]
    h = jnp.maximum(h, 0.0).astype(jnp.bfloat16)
    p = jnp.dot(h, w2_ref[...], preferred_element_type=jnp.float32) + b2_ref[...]
    return _l2n(p)


def _fused_kernel(nlab, lab_ref, txt_ref, tgt_ref, w1_ref, b1_ref, w2_ref,
                  b2_ref, out_ref, ln_ref):
    i = pl.program_id(0)
    lblk = lab_ref.shape[0]

    @pl.when(i < nlab)
    def _label_phase():
        ln = _proj_normed(lab_ref[...], w1_ref, b1_ref, w2_ref, b2_ref)
        ln_ref[pl.ds(i * lblk, lblk), :] = ln.astype(jnp.bfloat16)

    @pl.when(i >= nlab)
    def _text_phase():
        pn = _proj_normed(txt_ref[...], w1_ref, b1_ref, w2_ref,
                          b2_ref).astype(jnp.bfloat16)
        sim = jax.lax.dot_general(pn, ln_ref[...], (((1,), (1,)), ((), ())),
                                  preferred_element_type=jnp.float32)

        tgt = tgt_ref[...]
        pos = tgt > 0
        n_pos = jnp.sum(tgt, axis=-1, keepdims=True)
        masked = jnp.where(pos, NEG_FILL, sim)

        max_np = jnp.max(n_pos)
        t0 = jnp.full_like(n_pos, ABOVE, dtype=jnp.float32)
        thr0 = jnp.full_like(n_pos, BELOW, dtype=jnp.float32)

        def cond(carry):
            return carry[0] < max_np

        def body(carry):
            k, t, thr = carry
            m = jnp.max(jnp.where(masked < t, masked, BELOW), axis=-1,
                        keepdims=True)
            thr = jnp.where(k == n_pos - 1, m, thr)
            return k + 1, m, thr

        _, _, thr = jax.lax.while_loop(cond, body, (jnp.int32(0), t0, thr0))

        s = sim * INV_TEMP
        es = jnp.exp(s)
        sel = pos | (masked >= thr)
        denom = jnp.sum(jnp.where(sel, es, 0.0), axis=-1, keepdims=True)
        sum_pos_s = jnp.sum(jnp.where(pos, s, 0.0), axis=-1, keepdims=True)
        loss_rows = jnp.log(denom) - sum_pos_s / n_pos.astype(jnp.float32)
        out_ref[...] = jnp.zeros((1, 1, 128), jnp.float32) + jnp.sum(loss_rows)


def _full(shape):
    return pl.BlockSpec(shape, lambda *_: tuple(0 for _ in shape))


def kernel(text_embeddings, label_embeddings, target_labels, W1, b1, W2, b2):
    B, D = text_embeddings.shape
    L = label_embeddings.shape[0]
    H = W1.shape[1]
    blk = 512 if B % 512 == 0 else B
    lblk = L // 2 if L % 2 == 0 else L
    nblk = B // blk
    nlab = L // lblk

    w1b = W1.astype(jnp.bfloat16)
    w2b = W2.astype(jnp.bfloat16)
    b1r = b1.reshape(1, H)
    b2r = b2.reshape(1, D)

    partials = pl.pallas_call(
        functools.partial(_fused_kernel, nlab),
        grid=(nblk + nlab,),
        in_specs=[
            pl.BlockSpec((lblk, D), lambda i: (jnp.minimum(i, nlab - 1), 0)),
            pl.BlockSpec((blk, D), lambda i: (jnp.maximum(i - nlab, 0), 0)),
            pl.BlockSpec((blk, L), lambda i: (jnp.maximum(i - nlab, 0), 0)),
            _full((D, H)), _full((1, H)), _full((H, D)), _full((1, D)),
        ],
        out_specs=pl.BlockSpec((1, 1, 128),
                               lambda i: (jnp.maximum(i - nlab, 0), 0, 0)),
        out_shape=jax.ShapeDtypeStruct((nblk, 1, 128), jnp.float32),
        scratch_shapes=[pltpu.VMEM((L, D), jnp.bfloat16)],
        compiler_params=pltpu.CompilerParams(
            dimension_semantics=("arbitrary",),
            vmem_limit_bytes=52 * 1024 * 1024,
        ),
        name="tla_loss_fused",
    )(label_embeddings, text_embeddings, target_labels, w1b, b1r, w2b, b2r)

    return jnp.sum(partials[:, 0, 0]) / B
